# baseline (speedup 1.0000x reference)
import functools

import jax
import jax.numpy as jnp
from jax import lax
from jax.experimental import pallas as pl
from jax.experimental.pallas import tpu as pltpu
from jax.experimental.pallas import tpu_sc as plsc

_BM = 512
_BH = 1024

_NC = 2
_NS = 16
_NW = _NC * _NS
_L = 16


def _ffn_body(meta_ref, xs_ref, wg1_ref, wg2_ref, wl_ref, ws_ref, out_ref):
    w = pl.program_id(0)
    h = pl.program_id(1)
    tile = meta_ref[0, w]
    start = meta_ref[2, w]
    end = meta_ref[3, w]
    first = meta_ref[4, w]

    rows = tile * _BM + lax.broadcasted_iota(jnp.int32, (_BM, 1), 0)
    mask = (rows >= start) & (rows < end)
    xm = jnp.where(mask, xs_ref[...], 0.0).astype(jnp.bfloat16)

    x1 = lax.dot_general(xm, wg1_ref[0], (((1,), (1,)), ((), ())),
                         preferred_element_type=jnp.float32)
    x2 = lax.dot_general(xm, wg2_ref[0], (((1,), (1,)), ((), ())),
                         preferred_element_type=jnp.float32)
    act = (jax.nn.gelu(x1) * x2).astype(jnp.bfloat16)
    contrib = lax.dot_general(act, wl_ref[0], (((1,), (0,)), ((), ())),
                              preferred_element_type=jnp.float32)
    contrib = contrib * ws_ref[0, 0, :][:, None]

    init = (h == 0) & (first == 1)

    @pl.when(init)
    def _():
        out_ref[...] = contrib

    @pl.when(jnp.logical_not(init))
    def _():
        out_ref[...] = out_ref[...] + contrib


def _grouped_ffn(sorted_xs, wg1, wg2, wl, ws, meta, n_rows, feats, hidden):
    n_tiles = n_rows // _BM
    nh = hidden // _BH
    n_items = meta.shape[1]
    ws3 = ws.reshape(n_tiles, 1, _BM)
    grid_spec = pltpu.PrefetchScalarGridSpec(
        num_scalar_prefetch=1,
        grid=(n_items, nh),
        in_specs=[
            pl.BlockSpec((_BM, feats), lambda w, h, m: (m[0, w], 0)),
            pl.BlockSpec((1, _BH, feats), lambda w, h, m: (m[1, w], h, 0)),
            pl.BlockSpec((1, _BH, feats), lambda w, h, m: (m[1, w], h, 0)),
            pl.BlockSpec((1, _BH, feats), lambda w, h, m: (m[1, w], h, 0)),
            pl.BlockSpec((1, 1, _BM), lambda w, h, m: (m[0, w], 0, 0)),
        ],
        out_specs=pl.BlockSpec((_BM, feats), lambda w, h, m: (m[0, w], 0)),
    )
    return pl.pallas_call(
        _ffn_body,
        grid_spec=grid_spec,
        out_shape=jax.ShapeDtypeStruct((n_rows, feats), jnp.float32),
        compiler_params=pltpu.CompilerParams(
            dimension_semantics=("arbitrary", "arbitrary"),
        ),
    )(meta, sorted_xs, wg1, wg2, wl, ws3)


def _work_items(counts, n_rows, n_experts):
    n_tiles = n_rows // _BM
    n_items = n_tiles + n_experts - 1
    ends = jnp.cumsum(counts)
    starts = ends - counts
    first_tile = starts // _BM
    last_tile = jnp.maximum(ends - 1, 0) // _BM
    ntiles = jnp.where(counts > 0, last_tile - first_tile + 1, 0)
    cumw = jnp.cumsum(ntiles)
    total = cumw[-1]
    item_e = jnp.repeat(jnp.arange(n_experts), ntiles,
                        total_repeat_length=n_items)
    idx = jnp.arange(n_items)
    valid = idx < total
    off = idx - (cumw - ntiles)[item_e]
    tile_item = jnp.where(valid, first_tile[item_e] + off, n_tiles - 1)
    start_item = jnp.where(valid, starts[item_e], 0)
    end_item = jnp.where(valid, ends[item_e], 0)
    prev_tile = jnp.concatenate([jnp.full((1,), -1, tile_item.dtype),
                                 tile_item[:-1]])
    first_item = (tile_item != prev_tile).astype(jnp.int32)
    return jnp.stack([tile_item, item_e, start_item, end_item,
                      first_item]).astype(jnp.int32)


def _sc_dispatch(x2d, order, n_rows, feats):
    bpw = n_rows // _NW
    chunk = 64
    nch = bpw // chunk
    mesh = plsc.VectorSubcoreMesh(core_axis_name="c", subcore_axis_name="s")

    @functools.partial(
        pl.kernel,
        out_type=jax.ShapeDtypeStruct((n_rows, feats), jnp.float32),
        mesh=mesh,
        scratch_types=[
            pltpu.VMEM((bpw,), jnp.int32),
            pltpu.VMEM((bpw,), jnp.int32),
            pltpu.VMEM((chunk, feats), jnp.float32),
            pltpu.SemaphoreType.DMA,
        ],
    )
    def k(x_hbm, ord_hbm, out_hbm, idx_v, tok_v, rows_v, sem):
        wid = lax.axis_index("s") * _NC + lax.axis_index("c")
        base = wid * bpw
        pltpu.sync_copy(ord_hbm.at[pl.ds(base, bpw)], idx_v)
        for i in range(bpw // _L):
            v = idx_v[pl.ds(i * _L, _L)]
            tok_v[pl.ds(i * _L, _L)] = lax.shift_right_logical(v, 1)
        for c in range(nch):
            pltpu.async_copy(
                x_hbm.at[tok_v.at[pl.ds(c * chunk, chunk)]], rows_v, sem
            ).wait()
            pltpu.sync_copy(rows_v,
                            out_hbm.at[pl.ds(base + c * chunk, chunk)])

    return k(x2d, order)


def _sc_collect(y, slots_flat, n_tok, feats):
    tpw = n_tok // _NW
    tchunk = 32
    nch = tpw // tchunk
    mesh = plsc.VectorSubcoreMesh(core_axis_name="c", subcore_axis_name="s")

    @functools.partial(
        pl.kernel,
        out_type=jax.ShapeDtypeStruct((n_tok, feats), jnp.float32),
        mesh=mesh,
        scratch_types=[
            pltpu.VMEM((2 * tpw,), jnp.int32),
            pltpu.VMEM((2 * tchunk, feats), jnp.float32),
            pltpu.VMEM((tchunk, feats), jnp.float32),
            pltpu.SemaphoreType.DMA,
        ],
    )
    def k(y_hbm, sl_hbm, out_hbm, idx_v, rows_v, out_v, sem):
        wid = lax.axis_index("s") * _NC + lax.axis_index("c")
        base_t = wid * tpw
        pltpu.sync_copy(sl_hbm.at[pl.ds(2 * base_t, 2 * tpw)], idx_v)
        for c in range(nch):
            pltpu.async_copy(
                y_hbm.at[idx_v.at[pl.ds(c * 2 * tchunk, 2 * tchunk)]],
                rows_v, sem,
            ).wait()

            def tok_body(t, carry):
                for j in range(feats // _L):
                    sl = pl.ds(j * _L, _L)
                    out_v[t, sl] = rows_v[2 * t, sl] + rows_v[2 * t + 1, sl]
                return carry

            lax.fori_loop(0, tchunk, tok_body, 0)
            pltpu.sync_copy(out_v,
                            out_hbm.at[pl.ds(base_t + c * tchunk, tchunk)])

    return k(y, slots_flat)


def kernel(x, router_w, gating_w, linear_w, per_expert_scale, router_scale):
    g, s, feats = x.shape
    n_experts = router_w.shape[1]
    hidden = linear_w.shape[1]
    k = 2
    x2d = x.reshape(-1, feats)
    n_tok = x2d.shape[0]
    n_rows = n_tok * k

    var = jnp.mean(jnp.square(x2d), axis=-1, keepdims=True)
    ri = x2d * lax.rsqrt(var + 1e-6)
    ri = ri * lax.rsqrt(jnp.float32(feats)) * router_scale
    logits = ri @ router_w
    top_v, choices = lax.top_k(logits, k)
    cw = jax.nn.softmax(top_v, axis=-1)

    cflat = choices.reshape(-1)
    oh = (cflat[:, None] == jnp.arange(n_experts)[None, :]).astype(jnp.int32)
    csum = jnp.cumsum(oh, axis=0)
    counts = csum[-1]
    rank = jnp.take_along_axis(csum, cflat[:, None], axis=1)[:, 0] - 1
    starts = jnp.cumsum(counts) - counts
    inv = (starts[cflat] + rank).astype(jnp.int32)
    arange_rows = jnp.arange(n_rows, dtype=jnp.int32)
    order = jnp.zeros((n_rows,), jnp.int32).at[inv].set(arange_rows)
    meta = _work_items(counts, n_rows, n_experts)

    sorted_xs = _sc_dispatch(x2d, order, n_rows, feats)
    ws = jnp.zeros((n_rows,), cw.dtype).at[inv].set(cw.reshape(-1))

    wg1 = gating_w[:, 0].astype(jnp.bfloat16)
    wg2 = gating_w[:, 1].astype(jnp.bfloat16)
    wl = (linear_w * per_expert_scale[:, None, None]).astype(jnp.bfloat16)
    y = _grouped_ffn(sorted_xs, wg1, wg2, wl, ws, meta, n_rows, feats, hidden)

    out2d = _sc_collect(y, inv, n_tok, feats)
    return out2d.reshape(g, s, feats)


# speedup vs baseline: 1.0294x; 1.0294x over previous
import functools

import jax
import jax.numpy as jnp
from jax import lax
from jax.experimental import pallas as pl
from jax.experimental.pallas import tpu as pltpu
from jax.experimental.pallas import tpu_sc as plsc

_BM = 512
_BH = 1024

_NC = 2
_NS = 16
_NW = _NC * _NS
_L = 16


def _ffn_body(meta_ref, xs_ref, wg1_ref, wg2_ref, wl_ref, ws_ref, out_ref):
    w = pl.program_id(0)
    h = pl.program_id(1)
    tile = meta_ref[0, w]
    start = meta_ref[2, w]
    end = meta_ref[3, w]
    first = meta_ref[4, w]

    rows = tile * _BM + lax.broadcasted_iota(jnp.int32, (_BM, 1), 0)
    mask = (rows >= start) & (rows < end)
    xm = jnp.where(mask, xs_ref[...], 0.0).astype(jnp.bfloat16)

    x1 = lax.dot_general(xm, wg1_ref[0], (((1,), (1,)), ((), ())),
                         preferred_element_type=jnp.float32)
    x2 = lax.dot_general(xm, wg2_ref[0], (((1,), (1,)), ((), ())),
                         preferred_element_type=jnp.float32)
    act = (jax.nn.gelu(x1) * x2).astype(jnp.bfloat16)
    contrib = lax.dot_general(act, wl_ref[0], (((1,), (0,)), ((), ())),
                              preferred_element_type=jnp.float32)
    contrib = contrib * ws_ref[0, 0, :][:, None]

    init = (h == 0) & (first == 1)

    @pl.when(init)
    def _():
        out_ref[...] = contrib

    @pl.when(jnp.logical_not(init))
    def _():
        out_ref[...] = out_ref[...] + contrib


def _grouped_ffn(sorted_xs, wg1, wg2, wl, ws, meta, n_rows, feats, hidden):
    n_tiles = n_rows // _BM
    nh = hidden // _BH
    n_items = meta.shape[1]
    ws3 = ws.reshape(n_tiles, 1, _BM)
    grid_spec = pltpu.PrefetchScalarGridSpec(
        num_scalar_prefetch=1,
        grid=(n_items, nh),
        in_specs=[
            pl.BlockSpec((_BM, feats), lambda w, h, m: (m[0, w], 0)),
            pl.BlockSpec((1, _BH, feats), lambda w, h, m: (m[1, w], h, 0)),
            pl.BlockSpec((1, _BH, feats), lambda w, h, m: (m[1, w], h, 0)),
            pl.BlockSpec((1, _BH, feats), lambda w, h, m: (m[1, w], h, 0)),
            pl.BlockSpec((1, 1, _BM), lambda w, h, m: (m[0, w], 0, 0)),
        ],
        out_specs=pl.BlockSpec((_BM, feats), lambda w, h, m: (m[0, w], 0)),
    )
    return pl.pallas_call(
        _ffn_body,
        grid_spec=grid_spec,
        out_shape=jax.ShapeDtypeStruct((n_rows, feats), jnp.float32),
        compiler_params=pltpu.CompilerParams(
            dimension_semantics=("arbitrary", "arbitrary"),
        ),
    )(meta, sorted_xs, wg1, wg2, wl, ws3)


def _work_items(counts, n_rows, n_experts):
    n_tiles = n_rows // _BM
    n_items = n_tiles + n_experts - 1
    ends = jnp.cumsum(counts)
    starts = ends - counts
    first_tile = starts // _BM
    last_tile = jnp.maximum(ends - 1, 0) // _BM
    ntiles = jnp.where(counts > 0, last_tile - first_tile + 1, 0)
    cumw = jnp.cumsum(ntiles)
    total = cumw[-1]
    item_e = jnp.repeat(jnp.arange(n_experts), ntiles,
                        total_repeat_length=n_items)
    idx = jnp.arange(n_items)
    valid = idx < total
    off = idx - (cumw - ntiles)[item_e]
    tile_item = jnp.where(valid, first_tile[item_e] + off, n_tiles - 1)
    start_item = jnp.where(valid, starts[item_e], 0)
    end_item = jnp.where(valid, ends[item_e], 0)
    prev_tile = jnp.concatenate([jnp.full((1,), -1, tile_item.dtype),
                                 tile_item[:-1]])
    first_item = (tile_item != prev_tile).astype(jnp.int32)
    return jnp.stack([tile_item, item_e, start_item, end_item,
                      first_item]).astype(jnp.int32)


def _sc_dispatch(x2d, order, n_rows, feats):
    bpw = n_rows // _NW
    chunk = 64
    nch = bpw // chunk
    mesh = plsc.VectorSubcoreMesh(core_axis_name="c", subcore_axis_name="s")

    @functools.partial(
        pl.kernel,
        out_type=jax.ShapeDtypeStruct((n_rows, feats), jnp.float32),
        mesh=mesh,
        scratch_types=[
            pltpu.VMEM((bpw,), jnp.int32),
            pltpu.VMEM((bpw,), jnp.int32),
            pltpu.VMEM((chunk, feats), jnp.float32),
            pltpu.SemaphoreType.DMA,
        ],
    )
    def k(x_hbm, ord_hbm, out_hbm, idx_v, tok_v, rows_v, sem):
        wid = lax.axis_index("s") * _NC + lax.axis_index("c")
        base = wid * bpw
        pltpu.sync_copy(ord_hbm.at[pl.ds(base, bpw)], idx_v)
        for i in range(bpw // _L):
            v = idx_v[pl.ds(i * _L, _L)]
            tok_v[pl.ds(i * _L, _L)] = lax.shift_right_logical(v, 1)
        for c in range(nch):
            pltpu.async_copy(
                x_hbm.at[tok_v.at[pl.ds(c * chunk, chunk)]], rows_v, sem
            ).wait()
            pltpu.sync_copy(rows_v,
                            out_hbm.at[pl.ds(base + c * chunk, chunk)])

    return k(x2d, order)


def _sc_collect(y, slots_flat, n_tok, feats):
    tpw = n_tok // _NW
    tchunk = 32
    nch = tpw // tchunk
    mesh = plsc.VectorSubcoreMesh(core_axis_name="c", subcore_axis_name="s")

    @functools.partial(
        pl.kernel,
        out_type=jax.ShapeDtypeStruct((n_tok, feats), jnp.float32),
        mesh=mesh,
        scratch_types=[
            pltpu.VMEM((2 * tpw,), jnp.int32),
            pltpu.VMEM((2 * tchunk, feats), jnp.float32),
            pltpu.VMEM((tchunk, feats), jnp.float32),
            pltpu.SemaphoreType.DMA,
        ],
    )
    def k(y_hbm, sl_hbm, out_hbm, idx_v, rows_v, out_v, sem):
        wid = lax.axis_index("s") * _NC + lax.axis_index("c")
        base_t = wid * tpw
        pltpu.sync_copy(sl_hbm.at[pl.ds(2 * base_t, 2 * tpw)], idx_v)
        for c in range(nch):
            pltpu.async_copy(
                y_hbm.at[idx_v.at[pl.ds(c * 2 * tchunk, 2 * tchunk)]],
                rows_v, sem,
            ).wait()

            def tok_body(t, carry):
                for j in range(feats // _L):
                    sl = pl.ds(j * _L, _L)
                    out_v[t, sl] = rows_v[2 * t, sl] + rows_v[2 * t + 1, sl]
                return carry

            lax.fori_loop(0, tchunk, tok_body, 0)
            pltpu.sync_copy(out_v,
                            out_hbm.at[pl.ds(base_t + c * tchunk, tchunk)])

    return k(y, slots_flat)


def kernel(x, router_w, gating_w, linear_w, per_expert_scale, router_scale):
    g, s, feats = x.shape
    n_experts = router_w.shape[1]
    hidden = linear_w.shape[1]
    k = 2
    x2d = x.reshape(-1, feats)
    n_tok = x2d.shape[0]
    n_rows = n_tok * k

    var = jnp.mean(jnp.square(x2d), axis=-1, keepdims=True)
    ri = x2d * lax.rsqrt(var + 1e-6)
    ri = ri * lax.rsqrt(jnp.float32(feats)) * router_scale
    logits = ri @ router_w
    top_v, choices = lax.top_k(logits, k)
    cw = jax.nn.softmax(top_v, axis=-1)

    cflat = choices.reshape(-1)
    oh = (cflat[None, :] == jnp.arange(n_experts)[:, None]).astype(jnp.int32)
    csum = jnp.cumsum(oh, axis=1)
    counts = csum[:, -1]
    rank = jnp.sum(oh * csum, axis=0) - 1
    starts = jnp.cumsum(counts) - counts
    inv = (starts[cflat] + rank).astype(jnp.int32)
    arange_rows = jnp.arange(n_rows, dtype=jnp.int32)
    order = jnp.zeros((n_rows,), jnp.int32).at[inv].set(arange_rows)
    meta = _work_items(counts, n_rows, n_experts)

    sorted_xs = _sc_dispatch(x2d, order, n_rows, feats)
    ws = jnp.zeros((n_rows,), cw.dtype).at[inv].set(cw.reshape(-1))

    wg1 = gating_w[:, 0].astype(jnp.bfloat16)
    wg2 = gating_w[:, 1].astype(jnp.bfloat16)
    wl = (linear_w * per_expert_scale[:, None, None]).astype(jnp.bfloat16)
    y = _grouped_ffn(sorted_xs, wg1, wg2, wl, ws, meta, n_rows, feats, hidden)

    out2d = _sc_collect(y, inv, n_tok, feats)
    return out2d.reshape(g, s, feats)


# speedup vs baseline: 1.1094x; 1.0778x over previous
import functools

import jax
import jax.numpy as jnp
from jax import lax
from jax.experimental import pallas as pl
from jax.experimental.pallas import tpu as pltpu
from jax.experimental.pallas import tpu_sc as plsc

_BM = 512
_BH = 1024

_NC = 2
_NS = 16
_NW = _NC * _NS
_L = 16


def _ffn_body(meta_ref, xs_ref, wg1_ref, wg2_ref, wl_ref, ws_ref, out_ref):
    w = pl.program_id(0)
    h = pl.program_id(1)
    tile = meta_ref[0, w]
    start = meta_ref[2, w]
    end = meta_ref[3, w]
    first = meta_ref[4, w]

    rows = tile * _BM + lax.broadcasted_iota(jnp.int32, (_BM, 1), 0)
    mask = (rows >= start) & (rows < end)
    xm = jnp.where(mask, xs_ref[...], 0.0).astype(jnp.bfloat16)

    x1 = lax.dot_general(xm, wg1_ref[0], (((1,), (1,)), ((), ())),
                         preferred_element_type=jnp.float32)
    x2 = lax.dot_general(xm, wg2_ref[0], (((1,), (1,)), ((), ())),
                         preferred_element_type=jnp.float32)
    act = (jax.nn.gelu(x1) * x2).astype(jnp.bfloat16)
    contrib = lax.dot_general(act, wl_ref[0], (((1,), (0,)), ((), ())),
                              preferred_element_type=jnp.float32)
    contrib = contrib * ws_ref[0, 0, :][:, None]

    init = (h == 0) & (first == 1)

    @pl.when(init)
    def _():
        out_ref[...] = contrib

    @pl.when(jnp.logical_not(init))
    def _():
        out_ref[...] = out_ref[...] + contrib


def _grouped_ffn(sorted_xs, wg1, wg2, wl, ws, meta, n_rows, feats, hidden):
    n_tiles = n_rows // _BM
    nh = hidden // _BH
    n_items = meta.shape[1]
    ws3 = ws.reshape(n_tiles, 1, _BM)
    grid_spec = pltpu.PrefetchScalarGridSpec(
        num_scalar_prefetch=1,
        grid=(n_items, nh),
        in_specs=[
            pl.BlockSpec((_BM, feats), lambda w, h, m: (m[0, w], 0)),
            pl.BlockSpec((1, _BH, feats), lambda w, h, m: (m[1, w], h, 0)),
            pl.BlockSpec((1, _BH, feats), lambda w, h, m: (m[1, w], h, 0)),
            pl.BlockSpec((1, _BH, feats), lambda w, h, m: (m[1, w], h, 0)),
            pl.BlockSpec((1, 1, _BM), lambda w, h, m: (m[0, w], 0, 0)),
        ],
        out_specs=pl.BlockSpec((_BM, feats), lambda w, h, m: (m[0, w], 0)),
    )
    return pl.pallas_call(
        _ffn_body,
        grid_spec=grid_spec,
        out_shape=jax.ShapeDtypeStruct((n_rows, feats), jnp.float32),
        compiler_params=pltpu.CompilerParams(
            dimension_semantics=("arbitrary", "arbitrary"),
        ),
    )(meta, sorted_xs, wg1, wg2, wl, ws3)


def _work_items(counts, n_rows, n_experts):
    n_tiles = n_rows // _BM
    n_items = n_tiles + n_experts - 1
    ends = jnp.cumsum(counts)
    starts = ends - counts
    first_tile = starts // _BM
    last_tile = jnp.maximum(ends - 1, 0) // _BM
    ntiles = jnp.where(counts > 0, last_tile - first_tile + 1, 0)
    cumw = jnp.cumsum(ntiles)
    total = cumw[-1]
    item_e = jnp.repeat(jnp.arange(n_experts), ntiles,
                        total_repeat_length=n_items)
    idx = jnp.arange(n_items)
    valid = idx < total
    off = idx - (cumw - ntiles)[item_e]
    tile_item = jnp.where(valid, first_tile[item_e] + off, n_tiles - 1)
    start_item = jnp.where(valid, starts[item_e], 0)
    end_item = jnp.where(valid, ends[item_e], 0)
    prev_tile = jnp.concatenate([jnp.full((1,), -1, tile_item.dtype),
                                 tile_item[:-1]])
    first_item = (tile_item != prev_tile).astype(jnp.int32)
    return jnp.stack([tile_item, item_e, start_item, end_item,
                      first_item]).astype(jnp.int32)


def _sc_dispatch(x2d, order, n_rows, feats):
    bpw = n_rows // _NW
    chunk = 32
    nch = bpw // chunk
    mesh = plsc.VectorSubcoreMesh(core_axis_name="c", subcore_axis_name="s")

    @functools.partial(
        pl.kernel,
        out_type=jax.ShapeDtypeStruct((n_rows, feats), jnp.float32),
        mesh=mesh,
        scratch_types=[
            pltpu.VMEM((bpw,), jnp.int32),
            pltpu.VMEM((bpw,), jnp.int32),
            pltpu.VMEM((chunk, feats), jnp.float32),
            pltpu.VMEM((chunk, feats), jnp.float32),
            pltpu.SemaphoreType.DMA,
            pltpu.SemaphoreType.DMA,
        ],
    )
    def k(x_hbm, ord_hbm, out_hbm, idx_v, tok_v, rows_a, rows_b, sem_a, sem_b):
        wid = lax.axis_index("s") * _NC + lax.axis_index("c")
        base = wid * bpw
        pltpu.sync_copy(ord_hbm.at[pl.ds(base, bpw)], idx_v)
        for i in range(bpw // _L):
            v = idx_v[pl.ds(i * _L, _L)]
            tok_v[pl.ds(i * _L, _L)] = lax.shift_right_logical(v, 1)
        bufs = [(rows_a, sem_a), (rows_b, sem_b)]
        cps = [None, None]
        cps[0] = pltpu.async_copy(
            x_hbm.at[tok_v.at[pl.ds(0, chunk)]], rows_a, sem_a)
        for c in range(nch):
            cur = c % 2
            nxt = (c + 1) % 2
            if c + 1 < nch:
                cps[nxt] = pltpu.async_copy(
                    x_hbm.at[tok_v.at[pl.ds((c + 1) * chunk, chunk)]],
                    bufs[nxt][0], bufs[nxt][1])
            cps[cur].wait()
            pltpu.sync_copy(bufs[cur][0],
                            out_hbm.at[pl.ds(base + c * chunk, chunk)])

    return k(x2d, order)


def _sc_collect(y, slots_flat, n_tok, feats):
    tpw = n_tok // _NW
    tchunk = 16
    nch = tpw // tchunk
    mesh = plsc.VectorSubcoreMesh(core_axis_name="c", subcore_axis_name="s")

    @functools.partial(
        pl.kernel,
        out_type=jax.ShapeDtypeStruct((n_tok, feats), jnp.float32),
        mesh=mesh,
        scratch_types=[
            pltpu.VMEM((2 * tpw,), jnp.int32),
            pltpu.VMEM((2 * tchunk, feats), jnp.float32),
            pltpu.VMEM((2 * tchunk, feats), jnp.float32),
            pltpu.VMEM((tchunk, feats), jnp.float32),
            pltpu.SemaphoreType.DMA,
            pltpu.SemaphoreType.DMA,
        ],
    )
    def k(y_hbm, sl_hbm, out_hbm, idx_v, rows_a, rows_b, out_v,
          sem_a, sem_b):
        wid = lax.axis_index("s") * _NC + lax.axis_index("c")
        base_t = wid * tpw
        pltpu.sync_copy(sl_hbm.at[pl.ds(2 * base_t, 2 * tpw)], idx_v)
        bufs = [(rows_a, sem_a), (rows_b, sem_b)]
        cps = [None, None]
        cps[0] = pltpu.async_copy(
            y_hbm.at[idx_v.at[pl.ds(0, 2 * tchunk)]], rows_a, sem_a)
        for c in range(nch):
            cur = c % 2
            nxt = (c + 1) % 2
            if c + 1 < nch:
                cps[nxt] = pltpu.async_copy(
                    y_hbm.at[idx_v.at[pl.ds((c + 1) * 2 * tchunk,
                                            2 * tchunk)]],
                    bufs[nxt][0], bufs[nxt][1])
            cps[cur].wait()
            rows_v = bufs[cur][0]

            def tok_body(t, carry, rows_v=rows_v):
                for j in range(feats // _L):
                    sl = pl.ds(j * _L, _L)
                    out_v[t, sl] = rows_v[2 * t, sl] + rows_v[2 * t + 1, sl]
                return carry

            lax.fori_loop(0, tchunk, tok_body, 0)
            pltpu.sync_copy(out_v,
                            out_hbm.at[pl.ds(base_t + c * tchunk, tchunk)])

    return k(y, slots_flat)


def kernel(x, router_w, gating_w, linear_w, per_expert_scale, router_scale):
    g, s, feats = x.shape
    n_experts = router_w.shape[1]
    hidden = linear_w.shape[1]
    k = 2
    x2d = x.reshape(-1, feats)
    n_tok = x2d.shape[0]
    n_rows = n_tok * k

    var = jnp.mean(jnp.square(x2d), axis=-1, keepdims=True)
    ri = x2d * lax.rsqrt(var + 1e-6)
    ri = ri * lax.rsqrt(jnp.float32(feats)) * router_scale
    logits = ri @ router_w
    top_v, choices = lax.top_k(logits, k)
    cw = jax.nn.softmax(top_v, axis=-1)

    cflat = choices.reshape(-1)
    order = jnp.argsort(cflat, stable=True)
    inv = jnp.argsort(order).astype(jnp.int32)
    counts = jnp.sum(jax.nn.one_hot(cflat, n_experts, dtype=jnp.int32), axis=0)
    meta = _work_items(counts, n_rows, n_experts)

    sorted_xs = _sc_dispatch(x2d, order.astype(jnp.int32), n_rows, feats)
    ws = cw.reshape(-1)[order]

    wg1 = gating_w[:, 0].astype(jnp.bfloat16)
    wg2 = gating_w[:, 1].astype(jnp.bfloat16)
    wl = (linear_w * per_expert_scale[:, None, None]).astype(jnp.bfloat16)
    y = _grouped_ffn(sorted_xs, wg1, wg2, wl, ws, meta, n_rows, feats, hidden)

    out2d = _sc_collect(y, inv, n_tok, feats)
    return out2d.reshape(g, s, feats)


# speedup vs baseline: 1.1507x; 1.0372x over previous
import functools

import jax
import jax.numpy as jnp
from jax import lax
from jax.experimental import pallas as pl
from jax.experimental.pallas import tpu as pltpu
from jax.experimental.pallas import tpu_sc as plsc

_BM = 512
_BH = 2048

_NC = 2
_NS = 16
_NW = _NC * _NS
_L = 16


def _ffn_body(meta_ref, xs_ref, wg1_ref, wg2_ref, wl_ref, ws_ref, out_ref):
    w = pl.program_id(0)
    h = pl.program_id(1)
    tile = meta_ref[0, w]
    start = meta_ref[2, w]
    end = meta_ref[3, w]
    first = meta_ref[4, w]

    rows = tile * _BM + lax.broadcasted_iota(jnp.int32, (_BM, 1), 0)
    mask = (rows >= start) & (rows < end)
    xm = jnp.where(mask, xs_ref[...], 0.0).astype(jnp.bfloat16)

    x1 = lax.dot_general(xm, wg1_ref[0], (((1,), (1,)), ((), ())),
                         preferred_element_type=jnp.float32)
    x2 = lax.dot_general(xm, wg2_ref[0], (((1,), (1,)), ((), ())),
                         preferred_element_type=jnp.float32)
    act = (jax.nn.gelu(x1) * x2).astype(jnp.bfloat16)
    contrib = lax.dot_general(act, wl_ref[0], (((1,), (0,)), ((), ())),
                              preferred_element_type=jnp.float32)
    contrib = contrib * ws_ref[0, 0, :][:, None]

    init = (h == 0) & (first == 1)

    @pl.when(init)
    def _():
        out_ref[...] = contrib

    @pl.when(jnp.logical_not(init))
    def _():
        out_ref[...] = out_ref[...] + contrib


def _grouped_ffn(sorted_xs, wg1, wg2, wl, ws, meta, n_rows, feats, hidden):
    n_tiles = n_rows // _BM
    nh = hidden // _BH
    n_items = meta.shape[1]
    ws3 = ws.reshape(n_tiles, 1, _BM)
    grid_spec = pltpu.PrefetchScalarGridSpec(
        num_scalar_prefetch=1,
        grid=(n_items, nh),
        in_specs=[
            pl.BlockSpec((_BM, feats), lambda w, h, m: (m[0, w], 0)),
            pl.BlockSpec((1, _BH, feats), lambda w, h, m: (m[1, w], h, 0)),
            pl.BlockSpec((1, _BH, feats), lambda w, h, m: (m[1, w], h, 0)),
            pl.BlockSpec((1, _BH, feats), lambda w, h, m: (m[1, w], h, 0)),
            pl.BlockSpec((1, 1, _BM), lambda w, h, m: (m[0, w], 0, 0)),
        ],
        out_specs=pl.BlockSpec((_BM, feats), lambda w, h, m: (m[0, w], 0)),
    )
    return pl.pallas_call(
        _ffn_body,
        grid_spec=grid_spec,
        out_shape=jax.ShapeDtypeStruct((n_rows, feats), jnp.float32),
        compiler_params=pltpu.CompilerParams(
            dimension_semantics=("arbitrary", "arbitrary"),
        ),
    )(meta, sorted_xs, wg1, wg2, wl, ws3)


def _work_items(counts, n_rows, n_experts):
    n_tiles = n_rows // _BM
    n_items = n_tiles + n_experts - 1
    ends = jnp.cumsum(counts)
    starts = ends - counts
    first_tile = starts // _BM
    last_tile = jnp.maximum(ends - 1, 0) // _BM
    ntiles = jnp.where(counts > 0, last_tile - first_tile + 1, 0)
    cumw = jnp.cumsum(ntiles)
    total = cumw[-1]
    item_e = jnp.repeat(jnp.arange(n_experts), ntiles,
                        total_repeat_length=n_items)
    idx = jnp.arange(n_items)
    valid = idx < total
    off = idx - (cumw - ntiles)[item_e]
    tile_item = jnp.where(valid, first_tile[item_e] + off, n_tiles - 1)
    start_item = jnp.where(valid, starts[item_e], 0)
    end_item = jnp.where(valid, ends[item_e], 0)
    prev_tile = jnp.concatenate([jnp.full((1,), -1, tile_item.dtype),
                                 tile_item[:-1]])
    first_item = (tile_item != prev_tile).astype(jnp.int32)
    return jnp.stack([tile_item, item_e, start_item, end_item,
                      first_item]).astype(jnp.int32)


def _sc_dispatch(x2d, order, n_rows, feats):
    bpw = n_rows // _NW
    chunk = 32
    nch = bpw // chunk
    mesh = plsc.VectorSubcoreMesh(core_axis_name="c", subcore_axis_name="s")

    @functools.partial(
        pl.kernel,
        out_type=jax.ShapeDtypeStruct((n_rows, feats), jnp.float32),
        mesh=mesh,
        scratch_types=[
            pltpu.VMEM((bpw,), jnp.int32),
            pltpu.VMEM((bpw,), jnp.int32),
            pltpu.VMEM((chunk, feats), jnp.float32),
            pltpu.VMEM((chunk, feats), jnp.float32),
            pltpu.SemaphoreType.DMA,
            pltpu.SemaphoreType.DMA,
        ],
    )
    def k(x_hbm, ord_hbm, out_hbm, idx_v, tok_v, rows_a, rows_b, sem_a, sem_b):
        wid = lax.axis_index("s") * _NC + lax.axis_index("c")
        base = wid * bpw
        pltpu.sync_copy(ord_hbm.at[pl.ds(base, bpw)], idx_v)
        for i in range(bpw // _L):
            v = idx_v[pl.ds(i * _L, _L)]
            tok_v[pl.ds(i * _L, _L)] = lax.shift_right_logical(v, 1)
        bufs = [(rows_a, sem_a), (rows_b, sem_b)]
        cps = [None, None]
        cps[0] = pltpu.async_copy(
            x_hbm.at[tok_v.at[pl.ds(0, chunk)]], rows_a, sem_a)
        for c in range(nch):
            cur = c % 2
            nxt = (c + 1) % 2
            if c + 1 < nch:
                cps[nxt] = pltpu.async_copy(
                    x_hbm.at[tok_v.at[pl.ds((c + 1) * chunk, chunk)]],
                    bufs[nxt][0], bufs[nxt][1])
            cps[cur].wait()
            pltpu.sync_copy(bufs[cur][0],
                            out_hbm.at[pl.ds(base + c * chunk, chunk)])

    return k(x2d, order)


def _sc_collect(y, slots_flat, n_tok, feats):
    tpw = n_tok // _NW
    tchunk = 16
    nch = tpw // tchunk
    mesh = plsc.VectorSubcoreMesh(core_axis_name="c", subcore_axis_name="s")

    @functools.partial(
        pl.kernel,
        out_type=jax.ShapeDtypeStruct((n_tok, feats), jnp.float32),
        mesh=mesh,
        scratch_types=[
            pltpu.VMEM((2 * tpw,), jnp.int32),
            pltpu.VMEM((2 * tchunk, feats), jnp.float32),
            pltpu.VMEM((2 * tchunk, feats), jnp.float32),
            pltpu.VMEM((tchunk, feats), jnp.float32),
            pltpu.SemaphoreType.DMA,
            pltpu.SemaphoreType.DMA,
        ],
    )
    def k(y_hbm, sl_hbm, out_hbm, idx_v, rows_a, rows_b, out_v,
          sem_a, sem_b):
        wid = lax.axis_index("s") * _NC + lax.axis_index("c")
        base_t = wid * tpw
        pltpu.sync_copy(sl_hbm.at[pl.ds(2 * base_t, 2 * tpw)], idx_v)
        bufs = [(rows_a, sem_a), (rows_b, sem_b)]
        cps = [None, None]
        cps[0] = pltpu.async_copy(
            y_hbm.at[idx_v.at[pl.ds(0, 2 * tchunk)]], rows_a, sem_a)
        for c in range(nch):
            cur = c % 2
            nxt = (c + 1) % 2
            if c + 1 < nch:
                cps[nxt] = pltpu.async_copy(
                    y_hbm.at[idx_v.at[pl.ds((c + 1) * 2 * tchunk,
                                            2 * tchunk)]],
                    bufs[nxt][0], bufs[nxt][1])
            cps[cur].wait()
            rows_v = bufs[cur][0]

            def tok_body(t, carry, rows_v=rows_v):
                for j in range(feats // _L):
                    sl = pl.ds(j * _L, _L)
                    out_v[t, sl] = rows_v[2 * t, sl] + rows_v[2 * t + 1, sl]
                return carry

            lax.fori_loop(0, tchunk, tok_body, 0)
            pltpu.sync_copy(out_v,
                            out_hbm.at[pl.ds(base_t + c * tchunk, tchunk)])

    return k(y, slots_flat)


def kernel(x, router_w, gating_w, linear_w, per_expert_scale, router_scale):
    g, s, feats = x.shape
    n_experts = router_w.shape[1]
    hidden = linear_w.shape[1]
    k = 2
    x2d = x.reshape(-1, feats)
    n_tok = x2d.shape[0]
    n_rows = n_tok * k

    var = jnp.mean(jnp.square(x2d), axis=-1, keepdims=True)
    ri = x2d * lax.rsqrt(var + 1e-6)
    ri = ri * lax.rsqrt(jnp.float32(feats)) * router_scale
    logits = ri @ router_w
    top_v, choices = lax.top_k(logits, k)
    cw = jax.nn.softmax(top_v, axis=-1)

    cflat = choices.reshape(-1)
    order = jnp.argsort(cflat, stable=True)
    inv = jnp.argsort(order).astype(jnp.int32)
    counts = jnp.sum(jax.nn.one_hot(cflat, n_experts, dtype=jnp.int32), axis=0)
    meta = _work_items(counts, n_rows, n_experts)

    sorted_xs = _sc_dispatch(x2d, order.astype(jnp.int32), n_rows, feats)
    ws = cw.reshape(-1)[order]

    wg1 = gating_w[:, 0].astype(jnp.bfloat16)
    wg2 = gating_w[:, 1].astype(jnp.bfloat16)
    wl = (linear_w * per_expert_scale[:, None, None]).astype(jnp.bfloat16)
    y = _grouped_ffn(sorted_xs, wg1, wg2, wl, ws, meta, n_rows, feats, hidden)

    out2d = _sc_collect(y, inv, n_tok, feats)
    return out2d.reshape(g, s, feats)
